# baseline (speedup 1.0000x reference)
import functools

import jax
import jax.numpy as jnp
from jax import lax
from jax.experimental import pallas as pl
from jax.experimental.pallas import tpu as pltpu
from jax.experimental.pallas import tpu_sc as plsc

BATCH = 4096
EMBED_DIM = 256
NUM_VERTICES = 8192

_HB = BATCH // 2
_BK = 512
_NKB = NUM_VERTICES // _BK


def _make_sc_gather(dim, batch, dtype):
    info = plsc.get_sparse_core_info()
    nw = info.num_cores * info.num_subcores
    b_per_w = batch // nw
    mesh = plsc.VectorSubcoreMesh(core_axis_name="c", subcore_axis_name="s")

    @functools.partial(
        pl.kernel,
        mesh=mesh,
        out_type=jax.ShapeDtypeStruct((batch, dim), dtype),
        scratch_types=[
            pltpu.VMEM((b_per_w,), jnp.int32),
            pltpu.VMEM((b_per_w, dim), dtype),
            pltpu.SemaphoreType.DMA,
        ],
    )
    def gather(table_hbm, idx_hbm, out_hbm, idx_v, rows_v, sem):
        wid = lax.axis_index("s") * info.num_cores + lax.axis_index("c")
        base = wid * b_per_w
        pltpu.sync_copy(idx_hbm.at[pl.ds(base, b_per_w)], idx_v)
        pltpu.async_copy(table_hbm.at[idx_v], rows_v, sem).wait()
        pltpu.sync_copy(rows_v, out_hbm.at[pl.ds(base, b_per_w)])

    return gather


def _argmin_body(x_ref, v_ref, out_ref, mval_ref, midx_ref):
    j = pl.program_id(0)

    @pl.when(j == 0)
    def _():
        mval_ref[...] = jnp.full((_HB,), -jnp.inf, jnp.float32)
        midx_ref[...] = jnp.zeros((_HB,), jnp.float32)

    x = x_ref[...]
    v = v_ref[...]
    b2h = jnp.sum(v * v, axis=1, keepdims=True) * -0.5
    p = lax.dot_general(v, x, (((1,), (1,)), ((), ())),
                        preferred_element_type=jnp.float32) + b2h
    mj = jnp.max(p, axis=0)
    rows_i = lax.broadcasted_iota(jnp.int32, (_BK, _HB), 0) + (0x4B000000 + j * _BK)
    rows = lax.bitcast_convert_type(rows_i, jnp.float32)
    big = lax.bitcast_convert_type(jnp.int32(0x4B000000 + NUM_VERTICES), jnp.float32)
    ij = jnp.min(jnp.where(p >= mj[None, :], rows, big), axis=0)
    take = mj > mval_ref[...]
    mval_ref[...] = jnp.where(take, mj, mval_ref[...])
    midx_ref[...] = jnp.where(take, ij, midx_ref[...])

    @pl.when(j == _NKB - 1)
    def _():
        out_ref[0, 0, :] = (midx_ref[...] - jnp.float32(0x800000)).astype(jnp.int32)


_argmin_call = pl.pallas_call(
    _argmin_body,
    grid=(_NKB,),
    in_specs=[
        pl.BlockSpec((_HB, EMBED_DIM), lambda j: (0, 0)),
        pl.BlockSpec((_BK, EMBED_DIM), lambda j: (j, 0)),
    ],
    out_specs=pl.BlockSpec((1, 1, _HB), lambda j: (0, 0, 0)),
    out_shape=jax.ShapeDtypeStruct((1, 1, _HB), jnp.int32),
    scratch_shapes=[
        pltpu.VMEM((_HB,), jnp.float32),
        pltpu.VMEM((_HB,), jnp.float32),
    ],
)

_gather_raw = _make_sc_gather(EMBED_DIM, _HB, jnp.float32)
_gather_vertex = _make_sc_gather(EMBED_DIM, _HB, jnp.float32)


def kernel(concept_ids, raw_table, vertex_table):
    ids = concept_ids.astype(jnp.int32)
    init1 = _gather_raw(raw_table, ids[:_HB])
    init2 = _gather_raw(raw_table, ids[_HB:])
    n1 = _argmin_call(init1, vertex_table).reshape(_HB)
    n2 = _argmin_call(init2, vertex_table).reshape(_HB)
    f1 = _gather_vertex(vertex_table, n1)
    f2 = _gather_vertex(vertex_table, n2)
    return jnp.concatenate([f1, f2], axis=0)


# speedup vs baseline: 1.1313x; 1.1313x over previous
import functools

import jax
import jax.numpy as jnp
from jax import lax
from jax.experimental import pallas as pl
from jax.experimental.pallas import tpu as pltpu
from jax.experimental.pallas import tpu_sc as plsc

BATCH = 4096
EMBED_DIM = 256
NUM_VERTICES = 8192

_BK = 1024
_NKB = NUM_VERTICES // _BK


def _make_sc_gather(dim, batch, dtype):
    info = plsc.get_sparse_core_info()
    nw = info.num_cores * info.num_subcores
    b_per_w = batch // nw
    mesh = plsc.VectorSubcoreMesh(core_axis_name="c", subcore_axis_name="s")

    @functools.partial(
        pl.kernel,
        mesh=mesh,
        out_type=jax.ShapeDtypeStruct((batch, dim), dtype),
        scratch_types=[
            pltpu.VMEM((b_per_w,), jnp.int32),
            pltpu.VMEM((b_per_w, dim), dtype),
            pltpu.SemaphoreType.DMA,
        ],
    )
    def gather(table_hbm, idx_hbm, out_hbm, idx_v, rows_v, sem):
        wid = lax.axis_index("s") * info.num_cores + lax.axis_index("c")
        base = wid * b_per_w
        pltpu.sync_copy(idx_hbm.at[pl.ds(base, b_per_w)], idx_v)
        pltpu.async_copy(table_hbm.at[idx_v], rows_v, sem).wait()
        pltpu.sync_copy(rows_v, out_hbm.at[pl.ds(base, b_per_w)])

    return gather


def _argmin_body(x_ref, v_ref, out_ref, mval_ref, midx_ref):
    j = pl.program_id(0)

    @pl.when(j == 0)
    def _():
        mval_ref[...] = jnp.full((BATCH,), -jnp.inf, jnp.float32)
        midx_ref[...] = jnp.zeros((BATCH,), jnp.float32)

    x = x_ref[...]
    v = v_ref[...]
    b2h = jnp.sum(v * v, axis=1, keepdims=True) * -0.5
    p = lax.dot_general(v, x, (((1,), (1,)), ((), ())),
                        preferred_element_type=jnp.float32) + b2h
    mj = jnp.max(p, axis=0)
    rows_i = lax.broadcasted_iota(jnp.int32, (_BK, BATCH), 0) + (0x4B000000 + j * _BK)
    rows = lax.bitcast_convert_type(rows_i, jnp.float32)
    big = lax.bitcast_convert_type(jnp.int32(0x4B000000 + NUM_VERTICES), jnp.float32)
    ij = jnp.min(jnp.where(p >= mj[None, :], rows, big), axis=0)
    take = mj > mval_ref[...]
    mval_ref[...] = jnp.where(take, mj, mval_ref[...])
    midx_ref[...] = jnp.where(take, ij, midx_ref[...])

    @pl.when(j == _NKB - 1)
    def _():
        out_ref[0, 0, :] = (midx_ref[...] - jnp.float32(0x800000)).astype(jnp.int32)


_argmin_call = pl.pallas_call(
    _argmin_body,
    grid=(_NKB,),
    in_specs=[
        pl.BlockSpec((BATCH, EMBED_DIM), lambda j: (0, 0)),
        pl.BlockSpec((_BK, EMBED_DIM), lambda j: (j, 0)),
    ],
    out_specs=pl.BlockSpec((1, 1, BATCH), lambda j: (0, 0, 0)),
    out_shape=jax.ShapeDtypeStruct((1, 1, BATCH), jnp.int32),
    scratch_shapes=[
        pltpu.VMEM((BATCH,), jnp.float32),
        pltpu.VMEM((BATCH,), jnp.float32),
    ],
)

_gather_raw = _make_sc_gather(EMBED_DIM, BATCH, jnp.float32)
_gather_vertex = _make_sc_gather(EMBED_DIM, BATCH, jnp.float32)


def kernel(concept_ids, raw_table, vertex_table):
    ids = concept_ids.astype(jnp.int32)
    initial = _gather_raw(raw_table, ids)
    nearest = _argmin_call(initial, vertex_table).reshape(BATCH)
    final = _gather_vertex(vertex_table, nearest)
    return final
